# Initial kernel scaffold; baseline (speedup 1.0000x reference)
import jax
import jax.numpy as jnp
from jax.experimental import pallas as pl


def kernel(tokens, embedding):
    raise NotImplementedError("write your pallas kernel here")


# speedup vs baseline: 1.2871x; 1.2871x over previous
import functools

import jax
import jax.numpy as jnp
from jax import lax
from jax.experimental import pallas as pl
from jax.experimental.pallas import tpu as pltpu
from jax.experimental.pallas import tpu_sc as plsc

B = 4096 * 200
D = 128
NW = 32
BPW = B // NW
C = 128
STEPS = BPW // C

_mesh = plsc.VectorSubcoreMesh(core_axis_name="c", subcore_axis_name="s")


@functools.partial(
    pl.kernel,
    mesh=_mesh,
    out_type=jax.ShapeDtypeStruct((B, D), jnp.float32),
    scratch_types=[
        pltpu.VMEM((C,), jnp.int32),
        pltpu.VMEM((C, D), jnp.float32),
        pltpu.SemaphoreType.DMA,
    ],
)
def _emb_lookup(tokens_hbm, table_hbm, out_hbm, idx_v, rows_v, sem):
    wid = lax.axis_index("s") * 2 + lax.axis_index("c")
    base = wid * BPW

    def body(i, carry):
        off = base + i * C
        pltpu.sync_copy(tokens_hbm.at[pl.ds(off, C)], idx_v)
        pltpu.async_copy(table_hbm.at[idx_v], rows_v, sem).wait()
        pltpu.sync_copy(rows_v, out_hbm.at[pl.ds(off, C)])
        return carry

    lax.fori_loop(0, STEPS, body, 0)


def kernel(tokens, embedding):
    flat = tokens.reshape(-1).astype(jnp.int32)
    out = _emb_lookup(flat, embedding)
    return out.reshape(tokens.shape + (D,))


# speedup vs baseline: 1.3077x; 1.0160x over previous
import functools

import jax
import jax.numpy as jnp
from jax import lax
from jax.experimental import pallas as pl
from jax.experimental.pallas import tpu as pltpu
from jax.experimental.pallas import tpu_sc as plsc

B = 4096 * 200
D = 128
NW = 32
BPW = B // NW
C = 128
STEPS = BPW // C
NBUF = 4
NGROUPS = STEPS // NBUF

_mesh = plsc.VectorSubcoreMesh(core_axis_name="c", subcore_axis_name="s")


@functools.partial(
    pl.kernel,
    mesh=_mesh,
    out_type=jax.ShapeDtypeStruct((B, D), jnp.float32),
    scratch_types=(
        [
            pltpu.VMEM((STEPS, C), jnp.int32),
            pltpu.VMEM((NBUF, C, D), jnp.float32),
        ]
        + [pltpu.SemaphoreType.DMA for _ in range(2 * NBUF)]
    ),
)
def _emb_lookup(tokens_hbm, table_hbm, out_hbm, idx_v, rows_v, *sems):
    gsems = sems[:NBUF]
    wsems = sems[NBUF:]
    wid = lax.axis_index("s") * 2 + lax.axis_index("c")
    base = wid * BPW

    pltpu.sync_copy(tokens_hbm.at[pl.ds(wid * STEPS, STEPS)], idx_v)

    def group(g, carry):
        goff = base + g * (NBUF * C)
        for b in range(NBUF):
            @pl.when(g > 0)
            def _wait_prev_wb(b=b, goff=goff):
                pltpu.make_async_copy(
                    rows_v.at[b], out_hbm.at[pl.ds(goff, C)], wsems[b]
                ).wait()

            pltpu.async_copy(
                table_hbm.at[idx_v.at[g * NBUF + b]], rows_v.at[b], gsems[b]
            )
        for b in range(NBUF):
            pltpu.make_async_copy(
                table_hbm.at[idx_v.at[g * NBUF + b]], rows_v.at[b], gsems[b]
            ).wait()
            pltpu.async_copy(rows_v.at[b], out_hbm.at[pl.ds(goff + b * C, C)], wsems[b])
        return carry

    lax.fori_loop(0, NGROUPS, group, 0)

    for b in range(NBUF):
        pltpu.make_async_copy(
            rows_v.at[b], out_hbm.at[pl.ds(base, C)], wsems[b]
        ).wait()


def kernel(tokens, embedding):
    flat = tokens.reshape(-1, C).astype(jnp.int32)
    out = _emb_lookup(flat, embedding)
    return out.reshape(tokens.shape + (D,))


# speedup vs baseline: 15.6724x; 11.9848x over previous
import functools

import jax
import jax.numpy as jnp
from jax import lax
from jax.experimental import pallas as pl
from jax.experimental.pallas import tpu as pltpu
from jax.experimental.pallas import tpu_sc as plsc

B = 4096 * 200
D = 128
NW = 32
BPW = B // NW
C = 128
STEPS = BPW // C
NBUF = 4
NGROUPS = STEPS // NBUF

_mesh = plsc.VectorSubcoreMesh(core_axis_name="c", subcore_axis_name="s")


@functools.partial(
    pl.kernel,
    mesh=_mesh,
    out_type=jax.ShapeDtypeStruct((B, D), jnp.float32),
    scratch_types=(
        [
            pltpu.VMEM((STEPS, C), jnp.int32),
            pltpu.VMEM((NBUF, C, D), jnp.float32),
            pltpu.VMEM_SHARED((20, D), jnp.float32),
        ]
        + [pltpu.SemaphoreType.DMA for _ in range(2 * NBUF)]
    ),
)
def _emb_lookup(tokens_hbm, table_hbm, out_hbm, idx_v, rows_v, table_sh, *sems):
    gsems = sems[:NBUF]
    wsems = sems[NBUF:]
    sid = lax.axis_index("s")
    wid = sid * 2 + lax.axis_index("c")
    base = wid * BPW

    @pl.when(sid == 0)
    def _stage_table():
        pltpu.sync_copy(table_hbm, table_sh)

    pltpu.sync_copy(tokens_hbm.at[pl.ds(wid * STEPS, STEPS)], idx_v)
    plsc.subcore_barrier()

    def group(g, carry):
        goff = base + g * (NBUF * C)
        for b in range(NBUF):
            @pl.when(g > 0)
            def _wait_prev_wb(b=b, goff=goff):
                pltpu.make_async_copy(
                    rows_v.at[b], out_hbm.at[pl.ds(goff, C)], wsems[b]
                ).wait()

            pltpu.async_copy(
                table_sh.at[idx_v.at[g * NBUF + b]], rows_v.at[b], gsems[b]
            )
        for b in range(NBUF):
            pltpu.make_async_copy(
                table_sh.at[idx_v.at[g * NBUF + b]], rows_v.at[b], gsems[b]
            ).wait()
            pltpu.async_copy(rows_v.at[b], out_hbm.at[pl.ds(goff + b * C, C)], wsems[b])
        return carry

    lax.fori_loop(0, NGROUPS, group, 0)

    for b in range(NBUF):
        pltpu.make_async_copy(
            rows_v.at[b], out_hbm.at[pl.ds(base, C)], wsems[b]
        ).wait()


def kernel(tokens, embedding):
    flat = tokens.reshape(-1, C).astype(jnp.int32)
    out = _emb_lookup(flat, embedding)
    return out.reshape(tokens.shape + (D,))
